# baseline (device time: 117329 ns/iter reference)
import jax
import jax.numpy as jnp
from jax import lax
from jax.experimental import pallas as pl
from jax.experimental.pallas import tpu as pltpu

B = 2
S = 1024
S_HALF = S // 2
N = 2048


def kernel(O, Wo):
    b, s, h, d = O.shape
    o_flat = O.reshape(b, s, h * d)

    def body(o_ref, wo_ref, out_ref, send_buf, recv_buf, send_sem, recv_sem):
        my_x = lax.axis_index("x")
        my_y = lax.axis_index("y")
        peer = (1 - my_x, my_y)

        barrier_sem = pltpu.get_barrier_semaphore()
        pl.semaphore_signal(
            barrier_sem, inc=1,
            device_id=peer, device_id_type=pl.DeviceIdType.MESH,
        )
        pl.semaphore_wait(barrier_sem, 1)

        for bb in range(B):
            wo = wo_ref[:, :]
            out_ref[bb, :, :] = jnp.dot(
                o_ref[bb, pl.ds(my_x * S_HALF, S_HALF), :], wo,
                preferred_element_type=jnp.float32,
            )
            send_buf[bb, :, :] = jnp.dot(
                o_ref[bb, pl.ds((1 - my_x) * S_HALF, S_HALF), :], wo,
                preferred_element_type=jnp.float32,
            )

        rdma = pltpu.make_async_remote_copy(
            src_ref=send_buf,
            dst_ref=recv_buf,
            send_sem=send_sem,
            recv_sem=recv_sem,
            device_id=peer,
            device_id_type=pl.DeviceIdType.MESH,
        )
        rdma.start()
        rdma.wait()

        out_ref[:, :, :] += recv_buf[:, :, :]

    return pl.pallas_call(
        body,
        out_shape=jax.ShapeDtypeStruct((B, S_HALF, N), jnp.float32),
        in_specs=[
            pl.BlockSpec(memory_space=pltpu.VMEM),
            pl.BlockSpec(memory_space=pltpu.VMEM),
        ],
        out_specs=pl.BlockSpec(memory_space=pltpu.VMEM),
        scratch_shapes=[
            pltpu.VMEM((B, S_HALF, N), jnp.float32),
            pltpu.VMEM((B, S_HALF, N), jnp.float32),
            pltpu.SemaphoreType.DMA,
            pltpu.SemaphoreType.DMA,
        ],
        compiler_params=pltpu.CompilerParams(collective_id=0),
    )(o_flat, Wo)


# device time: 108065 ns/iter; 1.0857x vs baseline; 1.0857x over previous
import jax
import jax.numpy as jnp
from jax import lax
from jax.experimental import pallas as pl
from jax.experimental.pallas import tpu as pltpu

B = 2
S = 1024
S_HALF = S // 2
N = 2048

NC = 8
CHUNK = B * S_HALF // NC


def _chunk_bs(c):
    per_b = NC // B
    return c // per_b, (c % per_b) * CHUNK


def kernel(O, Wo):
    b, s, h, d = O.shape
    o_flat = O.reshape(b, s, h * d)

    def body(o_ref, wo_ref, out_ref, send_buf, recv_buf, send_sems, recv_sems):
        my_x = lax.axis_index("x")
        my_y = lax.axis_index("y")
        peer = (1 - my_x, my_y)

        barrier_sem = pltpu.get_barrier_semaphore()
        pl.semaphore_signal(
            barrier_sem, inc=1,
            device_id=peer, device_id_type=pl.DeviceIdType.MESH,
        )
        pl.semaphore_wait(barrier_sem, 1)

        wo = wo_ref[:, :]
        peer_off = (1 - my_x) * S_HALF
        my_off = my_x * S_HALF

        rdmas = []
        for c in range(NC):
            bb, off = _chunk_bs(c)
            send_buf[bb, pl.ds(off, CHUNK), :] = jnp.dot(
                o_ref[bb, pl.ds(peer_off + off, CHUNK), :], wo,
                preferred_element_type=jnp.float32,
            )
            rdma = pltpu.make_async_remote_copy(
                src_ref=send_buf.at[bb, pl.ds(off, CHUNK), :],
                dst_ref=recv_buf.at[bb, pl.ds(off, CHUNK), :],
                send_sem=send_sems.at[c],
                recv_sem=recv_sems.at[c],
                device_id=peer,
                device_id_type=pl.DeviceIdType.MESH,
            )
            rdma.start()
            rdmas.append(rdma)

        for bb in range(B):
            out_ref[bb, :, :] = jnp.dot(
                o_ref[bb, pl.ds(my_off, S_HALF), :], wo,
                preferred_element_type=jnp.float32,
            )

        for c in range(NC):
            bb, off = _chunk_bs(c)
            rdmas[c].wait_recv()
            out_ref[bb, pl.ds(off, CHUNK), :] += recv_buf[bb, pl.ds(off, CHUNK), :]
        for c in range(NC):
            rdmas[c].wait_send()

    return pl.pallas_call(
        body,
        out_shape=jax.ShapeDtypeStruct((B, S_HALF, N), jnp.float32),
        in_specs=[
            pl.BlockSpec(memory_space=pltpu.VMEM),
            pl.BlockSpec(memory_space=pltpu.VMEM),
        ],
        out_specs=pl.BlockSpec(memory_space=pltpu.VMEM),
        scratch_shapes=[
            pltpu.VMEM((B, S_HALF, N), jnp.float32),
            pltpu.VMEM((B, S_HALF, N), jnp.float32),
            pltpu.SemaphoreType.DMA((NC,)),
            pltpu.SemaphoreType.DMA((NC,)),
        ],
        compiler_params=pltpu.CompilerParams(collective_id=0),
    )(o_flat, Wo)
